# baseline (speedup 1.0000x reference)
import jax
import jax.numpy as jnp
from jax import lax
from jax.experimental import pallas as pl
from jax.experimental.pallas import tpu as pltpu
from jax.experimental.pallas import tpu_sc as plsc

N_FACTORS = 32
BATCH = 16384
N_ROWS = 1000000
NUM_CORES = 2
NUM_SUBCORES = 16
NW = NUM_CORES * NUM_SUBCORES
BPW = BATCH // NW
LANES = 16
NVREG = BATCH // LANES

CW = 1024
NCHUNK_FULL = N_ROWS // CW
TAIL_START = NCHUNK_FULL * CW
TAIL_ALIGNED = 512
EDGE_START = TAIL_START + TAIL_ALIGNED
EDGE_ROWS = N_ROWS - EDGE_START
CPT = -(-(NCHUNK_FULL + 1) // NW)
WLCAP = 1280
MAXM = 64
DUMP = BATCH
ROWP = 128


def _gather_body(idx_hbm, tab_hbm, edge_hbm, rows_hbm, idx_v, wl_idx,
                 wl_pos, chunk0, chunk1, m_idx, m_pos, edge_v, rows_v, sem,
                 csem0, csem1):
    tid = lax.axis_index("s") * NUM_CORES + lax.axis_index("c")

    pltpu.sync_copy(idx_hbm, idx_v)

    def extract(k, off):
        v = idx_v[pl.ds(k * LANES, LANES)]
        pos = k * LANES + lax.iota(jnp.int32, LANES)
        g = v // CW
        mine = lax.rem(g, NW) == tid
        tail = v >= TAIL_START
        m = jnp.where(tail, tid == NW - 1, mine)
        cnt = jnp.sum(m.astype(jnp.int32))
        plsc.store_compressed(wl_idx.at[pl.ds(off, LANES)], v, mask=m)
        plsc.store_compressed(wl_pos.at[pl.ds(off, LANES)], pos, mask=m)
        return jnp.minimum(off + cnt, WLCAP - LANES)

    nwl = lax.fori_loop(0, NVREG, extract, 0)
    nv = (nwl + LANES - 1) // LANES

    chunks = (chunk0, chunk1)
    csems = (csem0, csem1)

    def fire(ci, b):
        g = ci * NW + tid
        start = jnp.minimum(g, NCHUNK_FULL - 1) * CW
        for fg in range(N_FACTORS // 8):
            for lt in range(CW // 128):
                pltpu.async_copy(
                    tab_hbm.at[pl.ds(fg * 8, 8),
                               pl.ds(start + lt * 128, 128)],
                    chunks[b].at[fg * (CW // 128) + lt], csems[b])

    def wait_chunk(b):
        for t8 in range(N_FACTORS // 8 * (CW // 128)):
            pltpu.make_async_copy(tab_hbm.at[pl.ds(0, 8), pl.ds(0, 128)],
                                  chunks[b].at[t8], csems[b]).wait()

    def process(ci, chunk):
        g = ci * NW + tid
        start = jnp.minimum(g, NCHUNK_FULL - 1) * CW

        for q in range(MAXM // LANES):
            m_pos[pl.ds(q * LANES, LANES)] = jnp.full((LANES,), DUMP,
                                                      jnp.int32)

        def compact(k, off):
            v = wl_idx[pl.ds(k * LANES, LANES)]
            p = wl_pos[pl.ds(k * LANES, LANES)]
            live = (k * LANES + lax.iota(jnp.int32, LANES)) < nwl
            m = live & (v >= start) & (v < start + CW)
            cnt = jnp.sum(m.astype(jnp.int32))
            plsc.store_compressed(m_idx.at[pl.ds(off, LANES)], v, mask=m)
            plsc.store_compressed(m_pos.at[pl.ds(off, LANES)], p, mask=m)
            return jnp.minimum(off + cnt, MAXM - LANES)

        nm = lax.fori_loop(0, nv, compact, 0)

        def build(e, carry):
            ev = m_idx[pl.ds(e * LANES, LANES)]
            live = (e * LANES + lax.iota(jnp.int32, LANES)) < nm
            col = jnp.where(live, ev - start, 0)
            lt = lax.shift_right_logical(col, 7)
            ll = col & 127
            slot = e * LANES + lax.iota(jnp.int32, LANES)
            for f in range(N_FACTORS):
                vals = plsc.load_gather(
                    chunk, [(f // 8) * (CW // 128) + lt,
                            jnp.full((LANES,), f % 8, jnp.int32), ll])
                plsc.store_scatter(rows_v,
                                   [slot, jnp.full((LANES,), f, jnp.int32)],
                                   vals, mask=live)
            return carry

        nev = (nm + LANES - 1) // LANES
        lax.fori_loop(0, nev, build, 0)

        pltpu.async_copy(rows_v, rows_hbm.at[m_pos], sem).wait()

    fire(0, 0)

    def chunk_pair(ko, carry):
        for b in range(2):
            ci = ko * 2 + b

            @pl.when(ci < CPT)
            def _do(ci=ci, b=b):
                @pl.when(ci + 1 < CPT)
                def _prefetch(ci=ci, b=b):
                    fire(ci + 1, 1 - b)

                wait_chunk(b)
                process(ci, chunks[b])
        return carry

    lax.fori_loop(0, (CPT + 1) // 2, chunk_pair, 0)

    @pl.when(tid == NW - 1)
    def _tail():
        tcps = []
        for fg in range(N_FACTORS // 8):
            for lt in range(TAIL_ALIGNED // 128):
                tcps.append(pltpu.async_copy(
                    tab_hbm.at[pl.ds(fg * 8, 8),
                               pl.ds(TAIL_START + lt * 128, 128)],
                    chunks[0].at[fg * (CW // 128) + lt], csem0))
        for cp in tcps:
            cp.wait()
        pltpu.sync_copy(edge_hbm, edge_v)
        for q in range(MAXM // LANES):
            m_pos[pl.ds(q * LANES, LANES)] = jnp.full((LANES,), DUMP,
                                                      jnp.int32)

        def compact(k, off):
            v = wl_idx[pl.ds(k * LANES, LANES)]
            p = wl_pos[pl.ds(k * LANES, LANES)]
            live = (k * LANES + lax.iota(jnp.int32, LANES)) < nwl
            m = live & (v >= TAIL_START)
            cnt = jnp.sum(m.astype(jnp.int32))
            plsc.store_compressed(m_idx.at[pl.ds(off, LANES)], v, mask=m)
            plsc.store_compressed(m_pos.at[pl.ds(off, LANES)], p, mask=m)
            return jnp.minimum(off + cnt, MAXM - LANES)

        nm = lax.fori_loop(0, nv, compact, 0)

        def build(e, carry):
            ev = m_idx[pl.ds(e * LANES, LANES)]
            live = (e * LANES + lax.iota(jnp.int32, LANES)) < nm
            in_chunk = ev < EDGE_START
            ccol = jnp.where(live & in_chunk, ev - TAIL_START, 0)
            lt = lax.shift_right_logical(ccol, 7)
            ll = ccol & 127
            erow = jnp.where(live & ~in_chunk, ev - EDGE_START, 0)
            slot = e * LANES + lax.iota(jnp.int32, LANES)
            for f in range(N_FACTORS):
                vc = plsc.load_gather(
                    chunks[0], [(f // 8) * (CW // 128) + lt,
                                jnp.full((LANES,), f % 8, jnp.int32), ll])
                ve = plsc.load_gather(edge_v, [erow * N_FACTORS + f])
                vals = jnp.where(in_chunk, vc, ve)
                plsc.store_scatter(rows_v,
                                   [slot, jnp.full((LANES,), f, jnp.int32)],
                                   vals, mask=live)
            return carry

        nev = (nm + LANES - 1) // LANES
        lax.fori_loop(0, nev, build, 0)
        pltpu.async_copy(rows_v, rows_hbm.at[m_pos], sem).wait()


def _dot_body(uidx_hbm, iidx_hbm, urows_hbm, irows_hbm, ub_hbm, ib_hbm,
              out_hbm, uidx_v, iidx_v, ubuf0, ubuf1, ibuf0, ibuf1,
              ubv, ibv, outv, sem0, sem1, semb):
    wid = lax.axis_index("s") * NUM_CORES + lax.axis_index("c")
    base = wid * BPW

    pltpu.sync_copy(uidx_hbm.at[pl.ds(base, BPW)], uidx_v)
    pltpu.sync_copy(iidx_hbm.at[pl.ds(base, BPW)], iidx_v)
    bias_cps = []
    for c in range(4):
        bsl = pl.ds(c * 128, 128)
        bias_cps.append(pltpu.async_copy(ub_hbm.at[uidx_v.at[bsl]],
                                         ubv.at[bsl], semb))
        bias_cps.append(pltpu.async_copy(ib_hbm.at[iidx_v.at[bsl]],
                                         ibv.at[bsl], semb))

    ubufs, ibufs, sems = (ubuf0, ubuf1), (ibuf0, ibuf1), (sem0, sem1)

    def fire(c):
        s = sems[c % 2]
        sl = pl.ds(base + c * 128, 128)
        return (pltpu.async_copy(urows_hbm.at[sl], ubufs[c % 2], s),
                pltpu.async_copy(irows_hbm.at[sl], ibufs[c % 2], s))

    inflight = fire(0)
    for cp in bias_cps:
        cp.wait()
    for c in range(4):
        nxt = fire(c + 1) if c + 1 < 4 else None
        for cp in inflight:
            cp.wait()
        inflight = nxt
        ubuf, ibuf = ubufs[c % 2], ibufs[c % 2]

        def block(j, carry, ubuf=ubuf, ibuf=ibuf, c=c):
            b0 = c * 128 + j * LANES
            rows = j * LANES + lax.iota(jnp.int32, LANES)
            acc = ubv[pl.ds(b0, LANES)] + ibv[pl.ds(b0, LANES)]
            for f in range(N_FACTORS):
                uv = plsc.load_gather(ubuf,
                                      [rows, jnp.full((LANES,), f, jnp.int32)])
                iv = plsc.load_gather(ibuf,
                                      [rows, jnp.full((LANES,), f, jnp.int32)])
                acc = acc + uv * iv
            outv[pl.ds(b0, LANES)] = acc
            return carry

        lax.fori_loop(0, 128 // LANES, block, 0)

    pltpu.sync_copy(outv, out_hbm.at[pl.ds(base, BPW)])


def kernel(users_index, items_index, user_emb, item_emb, ub, ib):
    ut = user_emb.T
    it = item_emb.T
    ubf = ub.reshape(-1)
    ibf = ib.reshape(-1)
    uidx = users_index.astype(jnp.int32)
    iidx = items_index.astype(jnp.int32)

    mesh = plsc.VectorSubcoreMesh(core_axis_name="c", subcore_axis_name="s")
    cparams = pltpu.CompilerParams(needs_layout_passes=False)

    gather = pl.kernel(
        _gather_body,
        mesh=mesh,
        out_type=jax.ShapeDtypeStruct((BATCH + LANES, ROWP), jnp.float32),
        scratch_types=[
            pltpu.VMEM((BATCH,), jnp.int32),
            pltpu.VMEM((WLCAP,), jnp.int32),
            pltpu.VMEM((WLCAP,), jnp.int32),
            pltpu.VMEM((N_FACTORS // 8 * (CW // 128), 8, 128),
                       jnp.float32),
            pltpu.VMEM((N_FACTORS // 8 * (CW // 128), 8, 128),
                       jnp.float32),
            pltpu.VMEM((MAXM,), jnp.int32),
            pltpu.VMEM((MAXM,), jnp.int32),
            pltpu.VMEM((EDGE_ROWS * N_FACTORS,), jnp.float32),
            pltpu.VMEM((MAXM, ROWP), jnp.float32),
            pltpu.SemaphoreType.DMA,
            pltpu.SemaphoreType.DMA,
            pltpu.SemaphoreType.DMA,
        ],
        compiler_params=cparams,
    )
    edge_u = user_emb[EDGE_START:].reshape(-1)
    edge_i = item_emb[EDGE_START:].reshape(-1)
    urows = gather(uidx, ut, edge_u)
    irows = gather(iidx, it, edge_i)

    dot = pl.kernel(
        _dot_body,
        mesh=mesh,
        out_type=jax.ShapeDtypeStruct((BATCH,), jnp.float32),
        scratch_types=[
            pltpu.VMEM((BPW,), jnp.int32),
            pltpu.VMEM((BPW,), jnp.int32),
            pltpu.VMEM((128, ROWP), jnp.float32),
            pltpu.VMEM((128, ROWP), jnp.float32),
            pltpu.VMEM((128, ROWP), jnp.float32),
            pltpu.VMEM((128, ROWP), jnp.float32),
            pltpu.VMEM((BPW,), jnp.float32),
            pltpu.VMEM((BPW,), jnp.float32),
            pltpu.VMEM((BPW,), jnp.float32),
            pltpu.SemaphoreType.DMA,
            pltpu.SemaphoreType.DMA,
            pltpu.SemaphoreType.DMA,
        ],
        compiler_params=cparams,
    )
    return dot(uidx, iidx, urows, irows, ubf, ibf)


# speedup vs baseline: 11.1839x; 11.1839x over previous
import jax
import jax.numpy as jnp
from jax import lax
from jax.experimental import pallas as pl
from jax.experimental.pallas import tpu as pltpu
from jax.experimental.pallas import tpu_sc as plsc

N_FACTORS = 32
BATCH = 16384
N_ROWS = 1000000
NUM_CORES = 2
NUM_SUBCORES = 16
NW = NUM_CORES * NUM_SUBCORES
BPW = BATCH // NW
LANES = 16
NBLK = BPW // LANES

LAST_TILE_BASE = (N_ROWS // 128 - 1) * 128
EDGE_START = N_ROWS - N_ROWS % 128
EDGE_ROWS = N_ROWS - EDGE_START


def _pmf_body(uidx_hbm, iidx_hbm, ut_hbm, it_hbm, ub_hbm, ib_hbm,
              eu_hbm, ei_hbm, out_hbm, uidx_v, iidx_v, tbuf, urows, irows,
              ubv, ibv, eu_v, ei_v, outv, dsem, bsem):
    wid = lax.axis_index("s") * NUM_CORES + lax.axis_index("c")
    base = wid * BPW

    pltpu.sync_copy(uidx_hbm.at[pl.ds(base, BPW)], uidx_v)
    pltpu.sync_copy(iidx_hbm.at[pl.ds(base, BPW)], iidx_v)
    bias_cps = []
    for c in range(BPW // 128):
        bsl = pl.ds(c * 128, 128)
        bias_cps.append(pltpu.async_copy(ub_hbm.at[uidx_v.at[bsl]],
                                         ubv.at[bsl], bsem))
        bias_cps.append(pltpu.async_copy(ib_hbm.at[iidx_v.at[bsl]],
                                         ibv.at[bsl], bsem))
    pltpu.sync_copy(eu_hbm, eu_v)
    pltpu.sync_copy(ei_hbm, ei_v)

    fio = lax.iota(jnp.int32, LANES)

    def block(k, carry):
        iu16 = uidx_v[pl.ds(k * LANES, LANES)]
        ii16 = iidx_v[pl.ds(k * LANES, LANES)]
        for half in range(2):
            cps = []
            for l in range(8):
                lane = half * 8 + l
                for t, iv16, tab in ((0, iu16, ut_hbm), (1, ii16, it_hbm)):
                    sv = iv16[lane]
                    b0 = jnp.minimum(sv & -128, LAST_TILE_BASE)
                    b0 = pl.multiple_of(b0, 128)
                    for fg in range(N_FACTORS // 8):
                        cps.append(pltpu.async_copy(
                            tab.at[pl.ds(fg * 8, 8), pl.ds(b0, 128)],
                            tbuf.at[t, l, fg], dsem))
            for cp in cps:
                cp.wait()
            for l in range(8):
                lane = half * 8 + l
                pos = k * LANES + lane
                for t, iv16, rows, ev in ((0, iu16, urows, eu_v),
                                          (1, ii16, irows, ei_v)):
                    sv = iv16[lane]
                    lcol = jnp.zeros((LANES,), jnp.int32) + (sv & 127)
                    is_edge = sv >= EDGE_START
                    erow = jnp.maximum(sv - EDGE_START, 0)
                    for piece in range(2):
                        fvec = piece * LANES + fio
                        vals = plsc.load_gather(
                            tbuf,
                            [jnp.full((LANES,), t, jnp.int32),
                             jnp.full((LANES,), l, jnp.int32),
                             lax.shift_right_logical(fvec, 3),
                             fvec & 7, lcol])
                        evals = plsc.load_gather(
                            ev, [erow * N_FACTORS + fvec])
                        vals = jnp.where(is_edge, evals, vals)
                        rows[pl.ds(pos * N_FACTORS + piece * LANES,
                                   LANES)] = vals
        return carry

    lax.fori_loop(0, NBLK, block, 0)

    for cp in bias_cps:
        cp.wait()

    def dot(j, carry):
        b0 = j * LANES
        rowstart = b0 * N_FACTORS + fio * N_FACTORS
        acc = ubv[pl.ds(b0, LANES)] + ibv[pl.ds(b0, LANES)]
        for f in range(N_FACTORS):
            uv = plsc.load_gather(urows, [rowstart + f])
            iv = plsc.load_gather(irows, [rowstart + f])
            acc = acc + uv * iv
        outv[pl.ds(b0, LANES)] = acc
        return carry

    lax.fori_loop(0, NBLK, dot, 0)

    pltpu.sync_copy(outv, out_hbm.at[pl.ds(base, BPW)])


def kernel(users_index, items_index, user_emb, item_emb, ub, ib):
    ut = user_emb.T
    it = item_emb.T
    ubf = ub.reshape(-1)
    ibf = ib.reshape(-1)
    edge_u = user_emb[EDGE_START:].reshape(-1)
    edge_i = item_emb[EDGE_START:].reshape(-1)
    uidx = users_index.astype(jnp.int32)
    iidx = items_index.astype(jnp.int32)

    mesh = plsc.VectorSubcoreMesh(core_axis_name="c", subcore_axis_name="s")

    run = pl.kernel(
        _pmf_body,
        mesh=mesh,
        out_type=jax.ShapeDtypeStruct((BATCH,), jnp.float32),
        scratch_types=[
            pltpu.VMEM((BPW,), jnp.int32),
            pltpu.VMEM((BPW,), jnp.int32),
            pltpu.VMEM((2, 8, N_FACTORS // 8, 8, 128),
                       jnp.float32),
            pltpu.VMEM((BPW * N_FACTORS,), jnp.float32),
            pltpu.VMEM((BPW * N_FACTORS,), jnp.float32),
            pltpu.VMEM((BPW,), jnp.float32),
            pltpu.VMEM((BPW,), jnp.float32),
            pltpu.VMEM((EDGE_ROWS * N_FACTORS,), jnp.float32),
            pltpu.VMEM((EDGE_ROWS * N_FACTORS,), jnp.float32),
            pltpu.VMEM((BPW,), jnp.float32),
            pltpu.SemaphoreType.DMA,
            pltpu.SemaphoreType.DMA,
        ],
        compiler_params=pltpu.CompilerParams(needs_layout_passes=False),
    )
    return run(uidx, iidx, ut, it, ubf, ibf, edge_u, edge_i)


# speedup vs baseline: 11.5025x; 1.0285x over previous
import jax
import jax.numpy as jnp
from jax import lax
from jax.experimental import pallas as pl
from jax.experimental.pallas import tpu as pltpu
from jax.experimental.pallas import tpu_sc as plsc

N_FACTORS = 32
BATCH = 16384
N_ROWS = 1000000
NUM_CORES = 2
NUM_SUBCORES = 16
NW = NUM_CORES * NUM_SUBCORES
BPW = BATCH // NW
LANES = 16
NBLK = BPW // LANES

LAST_TILE_BASE = (N_ROWS // 128 - 1) * 128
EDGE_START = N_ROWS - N_ROWS % 128
EDGE_ROWS = N_ROWS - EDGE_START


def _pmf_body(uidx_hbm, iidx_hbm, ut_hbm, it_hbm, ub_hbm, ib_hbm,
              eu_hbm, ei_hbm, out_hbm, uidx_v, iidx_v, tbuf, urows, irows,
              ubv, ibv, eu_v, ei_v, outv, dsem0, dsem1, bsem):
    wid = lax.axis_index("s") * NUM_CORES + lax.axis_index("c")
    base = wid * BPW
    dsems = (dsem0, dsem1)

    pltpu.sync_copy(uidx_hbm.at[pl.ds(base, BPW)], uidx_v.at[pl.ds(0, BPW)])
    pltpu.sync_copy(iidx_hbm.at[pl.ds(base, BPW)], iidx_v.at[pl.ds(0, BPW)])
    bias_cps = []
    for c in range(BPW // 128):
        bsl = pl.ds(c * 128, 128)
        bias_cps.append(pltpu.async_copy(ub_hbm.at[uidx_v.at[bsl]],
                                         ubv.at[bsl], bsem))
        bias_cps.append(pltpu.async_copy(ib_hbm.at[iidx_v.at[bsl]],
                                         ibv.at[bsl], bsem))
    pltpu.sync_copy(eu_hbm, eu_v)
    pltpu.sync_copy(ei_hbm, ei_v)

    uidx_v[pl.ds(BPW, LANES)] = jnp.zeros((LANES,), jnp.int32)
    iidx_v[pl.ds(BPW, LANES)] = jnp.zeros((LANES,), jnp.int32)

    fio = lax.iota(jnp.int32, LANES)

    def fire(iu16, ii16, q):
        for l in range(4):
            lane = q * 4 + l
            for t, iv16, tab in ((0, iu16, ut_hbm), (1, ii16, it_hbm)):
                sv = iv16[lane]
                b0 = jnp.minimum(sv & -128, LAST_TILE_BASE)
                b0 = pl.multiple_of(b0, 128)
                for fg in range(N_FACTORS // 8):
                    pltpu.async_copy(
                        tab.at[pl.ds(fg * 8, 8), pl.ds(b0, 128)],
                        tbuf.at[q % 2, t, l, fg], dsems[q % 2])

    def wait_quarter(par):
        for _ in range(32):
            pltpu.make_async_copy(ut_hbm.at[pl.ds(0, 8), pl.ds(0, 128)],
                                  tbuf.at[par, 0, 0, 0],
                                  dsems[par]).wait()

    iu0 = uidx_v[pl.ds(0, LANES)]
    ii0 = iidx_v[pl.ds(0, LANES)]
    fire(iu0, ii0, 0)

    def block(k, carry):
        iu16 = uidx_v[pl.ds(k * LANES, LANES)]
        ii16 = iidx_v[pl.ds(k * LANES, LANES)]
        iun = uidx_v[pl.ds((k + 1) * LANES, LANES)]
        iin = iidx_v[pl.ds((k + 1) * LANES, LANES)]
        for q in range(4):
            if q < 3:
                fire(iu16, ii16, q + 1)
            else:
                @pl.when(k + 1 < NBLK)
                def _pf():
                    fire(iun, iin, 0)
            wait_quarter(q % 2)
            for l in range(4):
                lane = q * 4 + l
                pos = k * LANES + lane
                for t, iv16, rows, ev in ((0, iu16, urows, eu_v),
                                          (1, ii16, irows, ei_v)):
                    sv = iv16[lane]
                    lcol = jnp.zeros((LANES,), jnp.int32) + (sv & 127)
                    is_edge = sv >= EDGE_START
                    erow = jnp.maximum(sv - EDGE_START, 0)
                    for piece in range(2):
                        fvec = piece * LANES + fio
                        vals = plsc.load_gather(
                            tbuf,
                            [jnp.full((LANES,), q % 2, jnp.int32),
                             jnp.full((LANES,), t, jnp.int32),
                             jnp.full((LANES,), l, jnp.int32),
                             lax.shift_right_logical(fvec, 3),
                             fvec & 7, lcol])
                        evals = plsc.load_gather(
                            ev, [erow * N_FACTORS + fvec])
                        vals = jnp.where(is_edge, evals, vals)
                        rows[pl.ds(pos * N_FACTORS + piece * LANES,
                                   LANES)] = vals
        return carry

    lax.fori_loop(0, NBLK, block, 0)

    for cp in bias_cps:
        cp.wait()

    def dot(j, carry):
        b0 = j * LANES
        rowstart = b0 * N_FACTORS + fio * N_FACTORS
        acc = ubv[pl.ds(b0, LANES)] + ibv[pl.ds(b0, LANES)]
        for f in range(N_FACTORS):
            uv = plsc.load_gather(urows, [rowstart + f])
            iv = plsc.load_gather(irows, [rowstart + f])
            acc = acc + uv * iv
        outv[pl.ds(b0, LANES)] = acc
        return carry

    lax.fori_loop(0, NBLK, dot, 0)

    pltpu.sync_copy(outv, out_hbm.at[pl.ds(base, BPW)])


def kernel(users_index, items_index, user_emb, item_emb, ub, ib):
    ut = user_emb.T
    it = item_emb.T
    ubf = ub.reshape(-1)
    ibf = ib.reshape(-1)
    edge_u = user_emb[EDGE_START:].reshape(-1)
    edge_i = item_emb[EDGE_START:].reshape(-1)
    uidx = users_index.astype(jnp.int32)
    iidx = items_index.astype(jnp.int32)

    mesh = plsc.VectorSubcoreMesh(core_axis_name="c", subcore_axis_name="s")

    run = pl.kernel(
        _pmf_body,
        mesh=mesh,
        out_type=jax.ShapeDtypeStruct((BATCH,), jnp.float32),
        scratch_types=[
            pltpu.VMEM((BPW + LANES,), jnp.int32),
            pltpu.VMEM((BPW + LANES,), jnp.int32),
            pltpu.VMEM((2, 2, 4, N_FACTORS // 8, 8, 128),
                       jnp.float32),
            pltpu.VMEM((BPW * N_FACTORS,), jnp.float32),
            pltpu.VMEM((BPW * N_FACTORS,), jnp.float32),
            pltpu.VMEM((BPW,), jnp.float32),
            pltpu.VMEM((BPW,), jnp.float32),
            pltpu.VMEM((EDGE_ROWS * N_FACTORS,), jnp.float32),
            pltpu.VMEM((EDGE_ROWS * N_FACTORS,), jnp.float32),
            pltpu.VMEM((BPW,), jnp.float32),
            pltpu.SemaphoreType.DMA,
            pltpu.SemaphoreType.DMA,
            pltpu.SemaphoreType.DMA,
        ],
        compiler_params=pltpu.CompilerParams(needs_layout_passes=False),
    )
    return run(uidx, iidx, ut, it, ubf, ibf, edge_u, edge_i)


# speedup vs baseline: 11.5528x; 1.0044x over previous
import jax
import jax.numpy as jnp
from jax import lax
from jax.experimental import pallas as pl
from jax.experimental.pallas import tpu as pltpu
from jax.experimental.pallas import tpu_sc as plsc

N_FACTORS = 32
BATCH = 16384
N_ROWS = 1000000
NUM_CORES = 2
NUM_SUBCORES = 16
NW = NUM_CORES * NUM_SUBCORES
BPW = BATCH // NW
LANES = 16
NBLK = BPW // LANES

LAST_TILE_BASE = (N_ROWS // 128 - 1) * 128
EDGE_START = N_ROWS - N_ROWS % 128
EDGE_ROWS = N_ROWS - EDGE_START


def _pmf_body(uidx_hbm, iidx_hbm, ut_hbm, it_hbm, ub_hbm, ib_hbm,
              eu_hbm, ei_hbm, out_hbm, uidx_v, iidx_v, tbuf, urows, irows,
              ubv, ibv, eu_v, ei_v, outv, dsem0, dsem1, bsem):
    wid = lax.axis_index("s") * NUM_CORES + lax.axis_index("c")
    base = wid * BPW
    dsems = (dsem0, dsem1)

    pltpu.sync_copy(uidx_hbm.at[pl.ds(base, BPW)], uidx_v.at[pl.ds(0, BPW)])
    pltpu.sync_copy(iidx_hbm.at[pl.ds(base, BPW)], iidx_v.at[pl.ds(0, BPW)])
    bias_cps = []
    for c in range(BPW // 128):
        bsl = pl.ds(c * 128, 128)
        bias_cps.append(pltpu.async_copy(ub_hbm.at[uidx_v.at[bsl]],
                                         ubv.at[bsl], bsem))
        bias_cps.append(pltpu.async_copy(ib_hbm.at[iidx_v.at[bsl]],
                                         ibv.at[bsl], bsem))
    pltpu.sync_copy(eu_hbm, eu_v)
    pltpu.sync_copy(ei_hbm, ei_v)

    uidx_v[pl.ds(BPW, LANES)] = jnp.zeros((LANES,), jnp.int32)
    iidx_v[pl.ds(BPW, LANES)] = jnp.zeros((LANES,), jnp.int32)

    fio = lax.iota(jnp.int32, LANES)

    def fire(iu16, ii16, q):
        for l in range(4):
            lane = q * 4 + l
            for t, iv16, tab in ((0, iu16, ut_hbm), (1, ii16, it_hbm)):
                sv = iv16[lane]
                b0 = jnp.minimum(sv & -128, LAST_TILE_BASE)
                b0 = pl.multiple_of(b0, 128)
                pltpu.async_copy(
                    tab.at[:, pl.ds(b0, 128)],
                    tbuf.at[q % 2, t, l].reshape(N_FACTORS, 128),
                    dsems[q % 2])

    def wait_quarter(par):
        for _ in range(8):
            pltpu.make_async_copy(ut_hbm.at[:, pl.ds(0, 128)],
                                  tbuf.at[par, 0, 0].reshape(N_FACTORS, 128),
                                  dsems[par]).wait()

    iu0 = uidx_v[pl.ds(0, LANES)]
    ii0 = iidx_v[pl.ds(0, LANES)]
    fire(iu0, ii0, 0)

    def block(k, carry):
        iu16 = uidx_v[pl.ds(k * LANES, LANES)]
        ii16 = iidx_v[pl.ds(k * LANES, LANES)]
        iun = uidx_v[pl.ds((k + 1) * LANES, LANES)]
        iin = iidx_v[pl.ds((k + 1) * LANES, LANES)]
        for q in range(4):
            if q < 3:
                fire(iu16, ii16, q + 1)
            else:
                @pl.when(k + 1 < NBLK)
                def _pf():
                    fire(iun, iin, 0)
            wait_quarter(q % 2)
            for l in range(4):
                lane = q * 4 + l
                pos = k * LANES + lane
                for t, iv16, rows, ev in ((0, iu16, urows, eu_v),
                                          (1, ii16, irows, ei_v)):
                    sv = iv16[lane]
                    lcol = jnp.zeros((LANES,), jnp.int32) + (sv & 127)
                    is_edge = sv >= EDGE_START
                    erow = jnp.maximum(sv - EDGE_START, 0)
                    for piece in range(2):
                        fvec = piece * LANES + fio
                        vals = plsc.load_gather(
                            tbuf,
                            [jnp.full((LANES,), q % 2, jnp.int32),
                             jnp.full((LANES,), t, jnp.int32),
                             jnp.full((LANES,), l, jnp.int32),
                             lax.shift_right_logical(fvec, 3),
                             fvec & 7, lcol])
                        evals = plsc.load_gather(
                            ev, [erow * N_FACTORS + fvec])
                        vals = jnp.where(is_edge, evals, vals)
                        rows[pl.ds(pos * N_FACTORS + piece * LANES,
                                   LANES)] = vals
        return carry

    lax.fori_loop(0, NBLK, block, 0)

    for cp in bias_cps:
        cp.wait()

    def dot(j, carry):
        b0 = j * LANES
        rowstart = b0 * N_FACTORS + fio * N_FACTORS
        acc = ubv[pl.ds(b0, LANES)] + ibv[pl.ds(b0, LANES)]
        for f in range(N_FACTORS):
            uv = plsc.load_gather(urows, [rowstart + f])
            iv = plsc.load_gather(irows, [rowstart + f])
            acc = acc + uv * iv
        outv[pl.ds(b0, LANES)] = acc
        return carry

    lax.fori_loop(0, NBLK, dot, 0)

    pltpu.sync_copy(outv, out_hbm.at[pl.ds(base, BPW)])


def kernel(users_index, items_index, user_emb, item_emb, ub, ib):
    ut = user_emb.T
    it = item_emb.T
    ubf = ub.reshape(-1)
    ibf = ib.reshape(-1)
    edge_u = user_emb[EDGE_START:].reshape(-1)
    edge_i = item_emb[EDGE_START:].reshape(-1)
    uidx = users_index.astype(jnp.int32)
    iidx = items_index.astype(jnp.int32)

    mesh = plsc.VectorSubcoreMesh(core_axis_name="c", subcore_axis_name="s")

    run = pl.kernel(
        _pmf_body,
        mesh=mesh,
        out_type=jax.ShapeDtypeStruct((BATCH,), jnp.float32),
        scratch_types=[
            pltpu.VMEM((BPW + LANES,), jnp.int32),
            pltpu.VMEM((BPW + LANES,), jnp.int32),
            pltpu.VMEM((2, 2, 4, N_FACTORS // 8, 8, 128),
                       jnp.float32),
            pltpu.VMEM((BPW * N_FACTORS,), jnp.float32),
            pltpu.VMEM((BPW * N_FACTORS,), jnp.float32),
            pltpu.VMEM((BPW,), jnp.float32),
            pltpu.VMEM((BPW,), jnp.float32),
            pltpu.VMEM((EDGE_ROWS * N_FACTORS,), jnp.float32),
            pltpu.VMEM((EDGE_ROWS * N_FACTORS,), jnp.float32),
            pltpu.VMEM((BPW,), jnp.float32),
            pltpu.SemaphoreType.DMA,
            pltpu.SemaphoreType.DMA,
            pltpu.SemaphoreType.DMA,
        ],
        compiler_params=pltpu.CompilerParams(needs_layout_passes=False),
    )
    return run(uidx, iidx, ut, it, ubf, ibf, edge_u, edge_i)
